# baseline (speedup 1.0000x reference)
import functools
import math

import jax
import jax.numpy as jnp
from jax import lax
from jax.experimental import pallas as pl
from jax.experimental.pallas import tpu as pltpu
from jax.experimental.pallas import tpu_sc as plsc

B, S, D = 2, 2048, 768
H, DK = 12, 64
DFF = 2048
E = 8
CAP = 320
NB, MD = 32, 128
N = B * S

NC, NS = 2, 16
NW = NC * NS
TPB = N // NW
BCAP = B * CAP
NSLOT = E * BCAP
TRASH0 = NSLOT
NROWS = NSLOT + NW

TR = 512
TQ = 512


def _ln_qkv_body(x_ref, w_ref, g_ref, o_ref):
    x = x_ref[...]
    var = jnp.mean(x * x, axis=-1, keepdims=True)
    xn = (x * lax.rsqrt(var + 1e-6)) * g_ref[...]
    o_ref[...] = jnp.dot(xn.astype(jnp.bfloat16), w_ref[...],
                         preferred_element_type=jnp.float32).astype(
                             jnp.bfloat16)


def _ln_qkv(x, wqkv, g):
    return pl.pallas_call(
        _ln_qkv_body,
        grid=(N // TR,),
        in_specs=[
            pl.BlockSpec((TR, D), lambda i: (i, 0)),
            pl.BlockSpec((D, 3 * H * DK), lambda i: (0, 0)),
            pl.BlockSpec((1, D), lambda i: (0, 0)),
        ],
        out_specs=pl.BlockSpec((TR, 3 * H * DK), lambda i: (i, 0)),
        out_shape=jax.ShapeDtypeStruct((N, 3 * H * DK), jnp.bfloat16),
    )(x, wqkv, g)


def _attn_body(q_ref, k_ref, v_ref, bias_ref, o_ref):
    for hh in range(2):
        sl = slice(hh * DK, (hh + 1) * DK)
        q = q_ref[:, sl]
        k = k_ref[:, sl]
        v = v_ref[:, sl]
        scores = lax.dot_general(q, k, (((1,), (1,)), ((), ())),
                                 preferred_element_type=jnp.float32)
        scores = scores + bias_ref[0, hh]
        m = jnp.max(scores, axis=-1, keepdims=True)
        p = jnp.exp(scores - m)
        l = jnp.sum(p, axis=-1, keepdims=True)
        o = jnp.dot(p.astype(jnp.bfloat16), v,
                    preferred_element_type=jnp.float32) / l
        o_ref[:, sl] = o.astype(jnp.bfloat16)


H2 = H // 2
NQ = S // TQ


def _attention(qkv, posb4):
    return pl.pallas_call(
        _attn_body,
        grid=(B, H2, NQ),
        in_specs=[
            pl.BlockSpec((TQ, 2 * DK), lambda b, h2, qi: (b * NQ + qi, h2)),
            pl.BlockSpec((S, 2 * DK), lambda b, h2, qi: (b, 6 + h2)),
            pl.BlockSpec((S, 2 * DK), lambda b, h2, qi: (b, 12 + h2)),
            pl.BlockSpec((1, 2, TQ, S), lambda b, h2, qi: (h2, 0, qi, 0)),
        ],
        out_specs=pl.BlockSpec((TQ, 2 * DK), lambda b, h2, qi: (b * NQ + qi, h2)),
        out_shape=jax.ShapeDtypeStruct((N, H * DK), jnp.bfloat16),
    )(qkv, qkv, qkv, posb4)


def _proj_router_body(ctx_ref, wo_ref, hs_ref, g_ref, wr_ref,
                      hid_ref, y_ref, mp_ref, idx_ref):
    attn_out = jnp.dot(ctx_ref[...], wo_ref[...],
                       preferred_element_type=jnp.float32)
    hidden = hs_ref[...] + attn_out
    hid_ref[...] = hidden
    var = jnp.mean(hidden * hidden, axis=-1, keepdims=True)
    y = (hidden * lax.rsqrt(var + 1e-6)) * g_ref[...]
    y_ref[...] = y
    logits = jnp.dot(y, wr_ref[...], preferred_element_type=jnp.float32)
    mx = jnp.max(logits, axis=-1, keepdims=True)
    ex = jnp.exp(logits - mx)
    probs = ex / jnp.sum(ex, axis=-1, keepdims=True)
    mp = jnp.max(probs, axis=-1, keepdims=True)
    mp_ref[...] = mp
    cols = lax.broadcasted_iota(jnp.int32, (TR, E), 1)
    idx = jnp.min(jnp.where(probs >= mp, cols, E), axis=-1, keepdims=True)
    idx_ref[...] = idx


def _proj_router(ctx2, wo, hs2, g, wr):
    return pl.pallas_call(
        _proj_router_body,
        grid=(N // TR,),
        in_specs=[
            pl.BlockSpec((TR, H * DK), lambda i: (i, 0)),
            pl.BlockSpec((H * DK, D), lambda i: (0, 0)),
            pl.BlockSpec((TR, D), lambda i: (i, 0)),
            pl.BlockSpec((1, D), lambda i: (0, 0)),
            pl.BlockSpec((D, E), lambda i: (0, 0)),
        ],
        out_specs=[
            pl.BlockSpec((TR, D), lambda i: (i, 0)),
            pl.BlockSpec((TR, D), lambda i: (i, 0)),
            pl.BlockSpec((TR, 1), lambda i: (i, 0)),
            pl.BlockSpec((TR, 1), lambda i: (i, 0)),
        ],
        out_shape=[
            jax.ShapeDtypeStruct((N, D), jnp.float32),
            jax.ShapeDtypeStruct((N, D), jnp.float32),
            jax.ShapeDtypeStruct((N, 1), jnp.float32),
            jax.ShapeDtypeStruct((N, 1), jnp.int32),
        ],
    )(ctx2, wo, hs2, g, wr)


def _slots_body(idx_ref, slot_ref):
    b = pl.program_id(0)
    i = idx_ref[0]
    erow = lax.broadcasted_iota(jnp.int32, (E, S), 0)
    oh = (jnp.broadcast_to(i, (E, S)) == erow).astype(jnp.float32)
    r = lax.broadcasted_iota(jnp.int32, (S, S), 0)
    c = lax.broadcasted_iota(jnp.int32, (S, S), 1)
    tri = (r <= c).astype(jnp.float32)
    prio = lax.dot_general(oh, tri, (((1,), (0,)), ((), ())),
                           preferred_element_type=jnp.float32)
    p_tok = jnp.sum(oh * prio, axis=0, keepdims=True)
    kept = p_tok <= float(CAP)
    p_i = p_tok.astype(jnp.int32)
    scol = lax.broadcasted_iota(jnp.int32, (1, S), 1)
    flat = b * S + scol
    slot = jnp.where(kept, i * BCAP + b * CAP + (p_i - 1),
                     TRASH0 + flat // TPB)
    slot_ref[0] = slot


def _slots(eidx_b1s):
    return pl.pallas_call(
        _slots_body,
        grid=(B,),
        in_specs=[pl.BlockSpec((1, 1, S), lambda b: (b, 0, 0))],
        out_specs=pl.BlockSpec((1, 1, S), lambda b: (b, 0, 0)),
        out_shape=jax.ShapeDtypeStruct((B, 1, S), jnp.int32),
    )(eidx_b1s)


@functools.lru_cache(maxsize=None)
def _sc_kernels():
    mesh = plsc.VectorSubcoreMesh(core_axis_name="c", subcore_axis_name="s",
                                  num_cores=NC, num_subcores=NS)
    scratch = [
        pltpu.VMEM((TPB,), jnp.int32),
        pltpu.VMEM((TPB, D), jnp.float32),
        pltpu.SemaphoreType.DMA,
    ]

    @functools.partial(
        pl.kernel,
        out_type=jax.ShapeDtypeStruct((NROWS, D), jnp.float32),
        mesh=mesh, scratch_types=scratch)
    def sc_dispatch(y_hbm, slot_hbm, buf_hbm, idx_v, rows_v, sem):
        wid = lax.axis_index("s") * NC + lax.axis_index("c")
        base = wid * TPB
        pltpu.sync_copy(slot_hbm.at[pl.ds(base, TPB)], idx_v)
        pltpu.sync_copy(y_hbm.at[pl.ds(base, TPB)], rows_v)
        pltpu.async_copy(rows_v, buf_hbm.at[idx_v], sem).wait()

    @functools.partial(
        pl.kernel,
        out_type=jax.ShapeDtypeStruct((N, D), jnp.float32),
        mesh=mesh, scratch_types=scratch)
    def sc_gather(buf_hbm, slot_hbm, out_hbm, idx_v, rows_v, sem):
        wid = lax.axis_index("s") * NC + lax.axis_index("c")
        base = wid * TPB
        pltpu.sync_copy(slot_hbm.at[pl.ds(base, TPB)], idx_v)
        pltpu.async_copy(buf_hbm.at[idx_v], rows_v, sem).wait()
        pltpu.sync_copy(rows_v, out_hbm.at[pl.ds(base, TPB)])

    return sc_dispatch, sc_gather


def _experts_body(x_ref, wi_ref, wo_ref, o_ref):
    h1 = jnp.maximum(
        jnp.dot(x_ref[...].astype(jnp.bfloat16), wi_ref[0],
                preferred_element_type=jnp.float32),
        0.0)
    o_ref[...] = jnp.dot(h1.astype(jnp.bfloat16), wo_ref[0],
                         preferred_element_type=jnp.float32)


def _experts(buf, e_wi, e_wo):
    return pl.pallas_call(
        _experts_body,
        grid=(E,),
        in_specs=[
            pl.BlockSpec((BCAP, D), lambda e: (e, 0)),
            pl.BlockSpec((1, D, DFF), lambda e: (e, 0, 0)),
            pl.BlockSpec((1, DFF, D), lambda e: (e, 0, 0)),
        ],
        out_specs=pl.BlockSpec((BCAP, D), lambda e: (e, 0)),
        out_shape=jax.ShapeDtypeStruct((NROWS, D), jnp.float32),
    )(buf, e_wi, e_wo)


def _combine_body(hid_ref, y_ref, g_ref, mp_ref, slot_ref, o_ref):
    kept = slot_ref[...] < NSLOT
    ff = jnp.where(kept, g_ref[...], y_ref[...])
    o_ref[...] = hid_ref[...] + mp_ref[...] * ff


def _combine(hidden, y, gathered, mp, slot_col):
    return pl.pallas_call(
        _combine_body,
        grid=(N // TR,),
        in_specs=[
            pl.BlockSpec((TR, D), lambda i: (i, 0)),
            pl.BlockSpec((TR, D), lambda i: (i, 0)),
            pl.BlockSpec((TR, D), lambda i: (i, 0)),
            pl.BlockSpec((TR, 1), lambda i: (i, 0)),
            pl.BlockSpec((TR, 1), lambda i: (i, 0)),
        ],
        out_specs=pl.BlockSpec((TR, D), lambda i: (i, 0)),
        out_shape=jax.ShapeDtypeStruct((N, D), jnp.float32),
    )(hidden, y, gathered, mp, slot_col)


def _position_bias(rel_bias):
    rp = jnp.arange(-(S - 1), S, dtype=jnp.int32)
    nb = NB // 2
    rb = (rp > 0).astype(jnp.int32) * nb
    rpa = jnp.abs(rp)
    max_exact = nb // 2
    is_small = rpa < max_exact
    large = max_exact + (
        jnp.log(jnp.maximum(rpa, 1).astype(jnp.float32) / max_exact)
        / math.log(MD / max_exact)
        * (nb - max_exact)
    ).astype(jnp.int32)
    large = jnp.minimum(large, nb - 1)
    buckets = rb + jnp.where(is_small, rpa, large)
    v = rel_bias[buckets]
    u = jnp.pad(v.T, ((0, 0), (0, 1)))
    g = jnp.broadcast_to(u[:, None, :], (H, S, 2 * S)).reshape(H, 2 * S * S)
    g = g[:, S - 1:S - 1 + S * (2 * S - 1)].reshape(H, S, 2 * S - 1)
    return g[:, :, :S]


def kernel(hidden_states, ln1_w, wq, wk, wv, wo_attn, rel_bias, ln2_w, wr,
           e_wi, e_wo):
    hs2 = hidden_states.reshape(N, D)
    wqkv = jnp.concatenate([wq, wk, wv], axis=1).astype(jnp.bfloat16)
    qkv = _ln_qkv(hs2, wqkv, ln1_w.reshape(1, D))
    posb4 = _position_bias(rel_bias).reshape(H2, 2, S, S)
    ctxo = _attention(qkv, posb4)
    hidden, y, mp, eidx = _proj_router(ctxo, wo_attn.astype(jnp.bfloat16),
                                       hs2, ln2_w.reshape(1, D), wr)
    slot = _slots(eidx.reshape(B, 1, S)).reshape(N)
    sc_dispatch, sc_gather = _sc_kernels()
    buf = sc_dispatch(y, slot)
    buf2 = _experts(buf, e_wi.astype(jnp.bfloat16), e_wo.astype(jnp.bfloat16))
    gathered = sc_gather(buf2, slot)
    out = _combine(hidden, y, gathered, mp, slot.reshape(N, 1))
    return out.reshape(B, S, D)


# speedup vs baseline: 15.6911x; 15.6911x over previous
import functools
import math

import jax
import jax.numpy as jnp
from jax import lax
from jax.experimental import pallas as pl
from jax.experimental.pallas import tpu as pltpu
from jax.experimental.pallas import tpu_sc as plsc

B, S, D = 2, 2048, 768
H, DK = 12, 64
DFF = 2048
E = 8
CAP = 320
NB, MD = 32, 128
N = B * S

NC, NS = 2, 16
NW = NC * NS
TPB = N // NW
BCAP = B * CAP
NSLOT = E * BCAP
TRASH0 = NSLOT
NROWS = NSLOT + NW

TR = 512
TQ = 512


def _ln_qkv_body(x_ref, w_ref, g_ref, o_ref):
    x = x_ref[...]
    var = jnp.mean(x * x, axis=-1, keepdims=True)
    xn = (x * lax.rsqrt(var + 1e-6)) * g_ref[...]
    o_ref[...] = jnp.dot(xn.astype(jnp.bfloat16), w_ref[...],
                         preferred_element_type=jnp.float32).astype(
                             jnp.bfloat16)


def _ln_qkv(x, wqkv, g):
    return pl.pallas_call(
        _ln_qkv_body,
        grid=(N // TR,),
        in_specs=[
            pl.BlockSpec((TR, D), lambda i: (i, 0)),
            pl.BlockSpec((D, 3 * H * DK), lambda i: (0, 0)),
            pl.BlockSpec((1, D), lambda i: (0, 0)),
        ],
        out_specs=pl.BlockSpec((TR, 3 * H * DK), lambda i: (i, 0)),
        out_shape=jax.ShapeDtypeStruct((N, 3 * H * DK), jnp.bfloat16),
    )(x, wqkv, g)


ND = 31
NQ = S // TQ
H2 = H // 2
RQ = TQ // 128


def _attn_body(q_ref, k_ref, v_ref, bias_ref, o_ref):
    qi = pl.program_id(2)
    for hh in range(2):
        sl = slice(hh * DK, (hh + 1) * DK)
        q = q_ref[:, sl]
        k = k_ref[:, sl]
        v = v_ref[:, sl]
        scores = lax.dot_general(q, k, (((1,), (1,)), ((), ())),
                                 preferred_element_type=jnp.float32)
        rows = []
        for a in range(RQ):
            ag = qi * RQ + a
            cols = [bias_ref[hh, ci - ag + (ND // 2)]
                    for ci in range(S // 128)]
            rows.append(jnp.concatenate(cols, axis=1))
        scores = scores + jnp.concatenate(rows, axis=0)
        m = jnp.max(scores, axis=-1, keepdims=True)
        p = jnp.exp(scores - m)
        l = jnp.sum(p, axis=-1, keepdims=True)
        o = jnp.dot(p.astype(jnp.bfloat16), v,
                    preferred_element_type=jnp.float32) / l
        o_ref[:, sl] = o.astype(jnp.bfloat16)


def _attention(qkv, bias_blocks):
    return pl.pallas_call(
        _attn_body,
        grid=(B, H2, NQ),
        in_specs=[
            pl.BlockSpec((TQ, 2 * DK), lambda b, h2, qi: (b * NQ + qi, h2)),
            pl.BlockSpec((S, 2 * DK), lambda b, h2, qi: (b, 6 + h2)),
            pl.BlockSpec((S, 2 * DK), lambda b, h2, qi: (b, 12 + h2)),
            pl.BlockSpec((2, ND, 128, 128), lambda b, h2, qi: (h2, 0, 0, 0)),
        ],
        out_specs=pl.BlockSpec((TQ, 2 * DK), lambda b, h2, qi: (b * NQ + qi, h2)),
        out_shape=jax.ShapeDtypeStruct((N, H * DK), jnp.bfloat16),
    )(qkv, qkv, qkv, bias_blocks)


def _bias_blocks_body(u_ref, o_ref):
    u2 = u_ref[0]
    p_iota = lax.broadcasted_iota(jnp.int32, (128, 256), 0)
    c = 127 - p_iota
    for d in range(ND):
        m = jnp.broadcast_to(u2[:, 128 * d:128 * d + 256], (128, 256))
        for k in range(7):
            t = 1 << k
            rolled = jnp.concatenate([m[:, t:], m[:, :t]], axis=1)
            m = jnp.where(((c >> k) & 1) == 1, rolled, m)
        o_ref[0, d] = m[:, :128]


def _bias_blocks(u):
    return pl.pallas_call(
        _bias_blocks_body,
        grid=(H,),
        in_specs=[pl.BlockSpec((1, 1, 2 * S), lambda h: (h, 0, 0))],
        out_specs=pl.BlockSpec((1, ND, 128, 128), lambda h: (h, 0, 0, 0)),
        out_shape=jax.ShapeDtypeStruct((H, ND, 128, 128), jnp.float32),
    )(u.reshape(H, 1, 2 * S))


def _proj_router_body(ctx_ref, wo_ref, hs_ref, g_ref, wr_ref,
                      hid_ref, y_ref, mp_ref, idx_ref):
    attn_out = jnp.dot(ctx_ref[...], wo_ref[...],
                       preferred_element_type=jnp.float32)
    hidden = hs_ref[...] + attn_out
    hid_ref[...] = hidden
    var = jnp.mean(hidden * hidden, axis=-1, keepdims=True)
    y = (hidden * lax.rsqrt(var + 1e-6)) * g_ref[...]
    y_ref[...] = y
    logits = jnp.dot(y, wr_ref[...], preferred_element_type=jnp.float32)
    mx = jnp.max(logits, axis=-1, keepdims=True)
    ex = jnp.exp(logits - mx)
    probs = ex / jnp.sum(ex, axis=-1, keepdims=True)
    mp = jnp.max(probs, axis=-1, keepdims=True)
    mp_ref[...] = mp
    cols = lax.broadcasted_iota(jnp.int32, (TR, E), 1)
    idx = jnp.min(jnp.where(probs >= mp, cols, E), axis=-1, keepdims=True)
    idx_ref[...] = idx


def _proj_router(ctx2, wo, hs2, g, wr):
    return pl.pallas_call(
        _proj_router_body,
        grid=(N // TR,),
        in_specs=[
            pl.BlockSpec((TR, H * DK), lambda i: (i, 0)),
            pl.BlockSpec((H * DK, D), lambda i: (0, 0)),
            pl.BlockSpec((TR, D), lambda i: (i, 0)),
            pl.BlockSpec((1, D), lambda i: (0, 0)),
            pl.BlockSpec((D, E), lambda i: (0, 0)),
        ],
        out_specs=[
            pl.BlockSpec((TR, D), lambda i: (i, 0)),
            pl.BlockSpec((TR, D), lambda i: (i, 0)),
            pl.BlockSpec((TR, 1), lambda i: (i, 0)),
            pl.BlockSpec((TR, 1), lambda i: (i, 0)),
        ],
        out_shape=[
            jax.ShapeDtypeStruct((N, D), jnp.float32),
            jax.ShapeDtypeStruct((N, D), jnp.float32),
            jax.ShapeDtypeStruct((N, 1), jnp.float32),
            jax.ShapeDtypeStruct((N, 1), jnp.int32),
        ],
    )(ctx2, wo, hs2, g, wr)


def _slots_body(idx_ref, slot_ref):
    b = pl.program_id(0)
    i = idx_ref[0]
    erow = lax.broadcasted_iota(jnp.int32, (E, S), 0)
    oh = (jnp.broadcast_to(i, (E, S)) == erow).astype(jnp.float32)
    r = lax.broadcasted_iota(jnp.int32, (S, S), 0)
    c = lax.broadcasted_iota(jnp.int32, (S, S), 1)
    tri = (r <= c).astype(jnp.float32)
    prio = lax.dot_general(oh, tri, (((1,), (0,)), ((), ())),
                           preferred_element_type=jnp.float32)
    p_tok = jnp.sum(oh * prio, axis=0, keepdims=True)
    kept = p_tok <= float(CAP)
    p_i = p_tok.astype(jnp.int32)
    scol = lax.broadcasted_iota(jnp.int32, (1, S), 1)
    flat = b * S + scol
    slot = jnp.where(kept, i * BCAP + b * CAP + (p_i - 1),
                     TRASH0 + flat // TPB)
    slot_ref[0] = slot


def _slots(eidx_b1s):
    return pl.pallas_call(
        _slots_body,
        grid=(B,),
        in_specs=[pl.BlockSpec((1, 1, S), lambda b: (b, 0, 0))],
        out_specs=pl.BlockSpec((1, 1, S), lambda b: (b, 0, 0)),
        out_shape=jax.ShapeDtypeStruct((B, 1, S), jnp.int32),
    )(eidx_b1s)


@functools.lru_cache(maxsize=None)
def _sc_kernels():
    mesh = plsc.VectorSubcoreMesh(core_axis_name="c", subcore_axis_name="s",
                                  num_cores=NC, num_subcores=NS)
    scratch = [
        pltpu.VMEM((TPB,), jnp.int32),
        pltpu.VMEM((TPB, D), jnp.float32),
        pltpu.SemaphoreType.DMA,
    ]

    @functools.partial(
        pl.kernel,
        out_type=jax.ShapeDtypeStruct((NROWS, D), jnp.float32),
        mesh=mesh, scratch_types=scratch)
    def sc_dispatch(y_hbm, slot_hbm, buf_hbm, idx_v, rows_v, sem):
        wid = lax.axis_index("s") * NC + lax.axis_index("c")
        base = wid * TPB
        pltpu.sync_copy(slot_hbm.at[pl.ds(base, TPB)], idx_v)
        pltpu.sync_copy(y_hbm.at[pl.ds(base, TPB)], rows_v)
        pltpu.async_copy(rows_v, buf_hbm.at[idx_v], sem).wait()

    @functools.partial(
        pl.kernel,
        out_type=jax.ShapeDtypeStruct((N, D), jnp.float32),
        mesh=mesh, scratch_types=scratch)
    def sc_gather(buf_hbm, slot_hbm, out_hbm, idx_v, rows_v, sem):
        wid = lax.axis_index("s") * NC + lax.axis_index("c")
        base = wid * TPB
        pltpu.sync_copy(slot_hbm.at[pl.ds(base, TPB)], idx_v)
        pltpu.async_copy(buf_hbm.at[idx_v], rows_v, sem).wait()
        pltpu.sync_copy(rows_v, out_hbm.at[pl.ds(base, TPB)])

    return sc_dispatch, sc_gather


def _experts_body(x_ref, wi_ref, wo_ref, o_ref):
    h1 = jnp.maximum(
        jnp.dot(x_ref[...].astype(jnp.bfloat16), wi_ref[0],
                preferred_element_type=jnp.float32),
        0.0)
    o_ref[...] = jnp.dot(h1.astype(jnp.bfloat16), wo_ref[0],
                         preferred_element_type=jnp.float32)


def _experts(buf, e_wi, e_wo):
    return pl.pallas_call(
        _experts_body,
        grid=(E,),
        in_specs=[
            pl.BlockSpec((BCAP, D), lambda e: (e, 0)),
            pl.BlockSpec((1, D, DFF), lambda e: (e, 0, 0)),
            pl.BlockSpec((1, DFF, D), lambda e: (e, 0, 0)),
        ],
        out_specs=pl.BlockSpec((BCAP, D), lambda e: (e, 0)),
        out_shape=jax.ShapeDtypeStruct((NROWS, D), jnp.float32),
    )(buf, e_wi, e_wo)


def _combine_body(hid_ref, y_ref, g_ref, mp_ref, slot_ref, o_ref):
    kept = slot_ref[...] < NSLOT
    ff = jnp.where(kept, g_ref[...], y_ref[...])
    o_ref[...] = hid_ref[...] + mp_ref[...] * ff


def _combine(hidden, y, gathered, mp, slot_col):
    return pl.pallas_call(
        _combine_body,
        grid=(N // TR,),
        in_specs=[
            pl.BlockSpec((TR, D), lambda i: (i, 0)),
            pl.BlockSpec((TR, D), lambda i: (i, 0)),
            pl.BlockSpec((TR, D), lambda i: (i, 0)),
            pl.BlockSpec((TR, 1), lambda i: (i, 0)),
            pl.BlockSpec((TR, 1), lambda i: (i, 0)),
        ],
        out_specs=pl.BlockSpec((TR, D), lambda i: (i, 0)),
        out_shape=jax.ShapeDtypeStruct((N, D), jnp.float32),
    )(hidden, y, gathered, mp, slot_col)


def _bias_diag_table(rel_bias):
    rp = jnp.arange(-(S - 1), S, dtype=jnp.int32)
    nb = NB // 2
    rb = (rp > 0).astype(jnp.int32) * nb
    rpa = jnp.abs(rp)
    max_exact = nb // 2
    is_small = rpa < max_exact
    large = max_exact + (
        jnp.log(jnp.maximum(rpa, 1).astype(jnp.float32) / max_exact)
        / math.log(MD / max_exact)
        * (nb - max_exact)
    ).astype(jnp.int32)
    large = jnp.minimum(large, nb - 1)
    buckets = rb + jnp.where(is_small, rpa, large)
    v = rel_bias[buckets]
    return jnp.pad(v.T, ((0, 0), (0, 1)))


def kernel(hidden_states, ln1_w, wq, wk, wv, wo_attn, rel_bias, ln2_w, wr,
           e_wi, e_wo):
    hs2 = hidden_states.reshape(N, D)
    wqkv = jnp.concatenate([wq, wk, wv], axis=1).astype(jnp.bfloat16)
    qkv = _ln_qkv(hs2, wqkv, ln1_w.reshape(1, D))
    bias_blocks = _bias_blocks(_bias_diag_table(rel_bias))
    ctxo = _attention(qkv, bias_blocks)
    hidden, y, mp, eidx = _proj_router(ctxo, wo_attn.astype(jnp.bfloat16),
                                       hs2, ln2_w.reshape(1, D), wr)
    slot = _slots(eidx.reshape(B, 1, S)).reshape(N)
    sc_dispatch, sc_gather = _sc_kernels()
    buf = sc_dispatch(y, slot)
    buf2 = _experts(buf, e_wi.astype(jnp.bfloat16), e_wo.astype(jnp.bfloat16))
    gathered = sc_gather(buf2, slot)
    out = _combine(hidden, y, gathered, mp, slot.reshape(N, 1))
    return out.reshape(B, S, D)
